# Initial kernel scaffold; baseline (speedup 1.0000x reference)
import jax
import jax.numpy as jnp
from jax.experimental import pallas as pl


def kernel(x_comp, u, cell_node_map):
    raise NotImplementedError("write your pallas kernel here")


# speedup vs baseline: 1615.9453x; 1615.9453x over previous
import functools

import jax
import jax.numpy as jnp
from jax import lax
from jax.experimental import pallas as pl
from jax.experimental.pallas import tpu as pltpu
from jax.experimental.pallas import tpu_sc as plsc

NC = 2
NS = 16
NW = NC * NS
L = 16
B = 128


def _sc_assemble(npad, nblk, chunk):
    mesh = plsc.VectorSubcoreMesh(core_axis_name="c", subcore_axis_name="s")
    wtri = nblk * B

    @functools.partial(
        pl.kernel,
        out_type=jax.ShapeDtypeStruct((NC * npad,), jnp.float32),
        mesh=mesh,
        scratch_types=dict(
            xs_sp=pltpu.VMEM_SHARED((npad,), jnp.float32),
            ys_sp=pltpu.VMEM_SHARED((npad,), jnp.float32),
            us_sp=pltpu.VMEM_SHARED((npad,), jnp.float32),
            acc_sp=pltpu.VMEM_SHARED((npad,), jnp.float32),
            ibig=pltpu.VMEM((wtri,), jnp.int32),
            jbig=pltpu.VMEM((wtri,), jnp.int32),
            kbig=pltpu.VMEM((wtri,), jnp.int32),
            iv=pltpu.VMEM((B,), jnp.int32),
            jv=pltpu.VMEM((B,), jnp.int32),
            kv=pltpu.VMEM((B,), jnp.int32),
            gxi=pltpu.VMEM((B,), jnp.float32),
            gxj=pltpu.VMEM((B,), jnp.float32),
            gxk=pltpu.VMEM((B,), jnp.float32),
            gyi=pltpu.VMEM((B,), jnp.float32),
            gyj=pltpu.VMEM((B,), jnp.float32),
            gyk=pltpu.VMEM((B,), jnp.float32),
            gui=pltpu.VMEM((B,), jnp.float32),
            guj=pltpu.VMEM((B,), jnp.float32),
            guk=pltpu.VMEM((B,), jnp.float32),
            vib=pltpu.VMEM((B,), jnp.float32),
            vjb=pltpu.VMEM((B,), jnp.float32),
            vkb=pltpu.VMEM((B,), jnp.float32),
            sem=pltpu.SemaphoreType.DMA,
        ),
    )
    def body(xs_h, ys_h, us_h, z_h, i_h, j_h, k_h, out_h, *,
             xs_sp, ys_sp, us_sp, acc_sp, ibig, jbig, kbig, iv, jv, kv,
             gxi, gxj, gxk, gyi, gyj, gyk, gui, guj, guk,
             vib, vjb, vkb, sem):
        c = lax.axis_index("c")
        s = lax.axis_index("s")
        wid = c * NS + s

        noff = s * chunk
        pltpu.sync_copy(xs_h.at[pl.ds(noff, chunk)], xs_sp.at[pl.ds(noff, chunk)])
        pltpu.sync_copy(ys_h.at[pl.ds(noff, chunk)], ys_sp.at[pl.ds(noff, chunk)])
        pltpu.sync_copy(us_h.at[pl.ds(noff, chunk)], us_sp.at[pl.ds(noff, chunk)])
        pltpu.sync_copy(z_h.at[pl.ds(noff, chunk)], acc_sp.at[pl.ds(noff, chunk)])
        toff = wid * wtri
        pltpu.sync_copy(i_h.at[pl.ds(toff, wtri)], ibig)
        pltpu.sync_copy(j_h.at[pl.ds(toff, wtri)], jbig)
        pltpu.sync_copy(k_h.at[pl.ds(toff, wtri)], kbig)
        plsc.subcore_barrier()

        def blk(b, carry):
            boff = b * B
            for t in range(B // L):
                sl = pl.ds(t * L, L)
                bsl = pl.ds(boff + t * L, L)
                iv[sl] = ibig[bsl]
                jv[sl] = jbig[bsl]
                kv[sl] = kbig[bsl]
            cps = [
                pltpu.async_copy(xs_sp.at[iv], gxi, sem),
                pltpu.async_copy(xs_sp.at[jv], gxj, sem),
                pltpu.async_copy(xs_sp.at[kv], gxk, sem),
                pltpu.async_copy(ys_sp.at[iv], gyi, sem),
                pltpu.async_copy(ys_sp.at[jv], gyj, sem),
                pltpu.async_copy(ys_sp.at[kv], gyk, sem),
                pltpu.async_copy(us_sp.at[iv], gui, sem),
                pltpu.async_copy(us_sp.at[jv], guj, sem),
                pltpu.async_copy(us_sp.at[kv], guk, sem),
            ]
            for cp in cps:
                cp.wait()
            for t in range(B // L):
                sl = pl.ds(t * L, L)
                xi = gxi[sl]
                xj = gxj[sl]
                xk = gxk[sl]
                yi = gyi[sl]
                yj = gyj[sl]
                yk = gyk[sl]
                ui = gui[sl]
                uj = guj[sl]
                uk = guk[sl]
                nix = yj - yk
                niy = xk - xj
                njx = yk - yi
                njy = xi - xk
                nkx = yi - yj
                nky = xj - xi
                det = njy * nix - niy * njx
                sc = 0.5 / jnp.abs(det)
                mii = (nix * nix + niy * niy) * sc
                mjj = (njx * njx + njy * njy) * sc
                mkk = (nkx * nkx + nky * nky) * sc
                mij = (nix * njx + niy * njy) * sc
                mjk = (njx * nkx + njy * nky) * sc
                mki = (nkx * nix + nky * niy) * sc
                vib[sl] = -(mii * ui + mij * uj + mki * uk)
                vjb[sl] = -(mij * ui + mjj * uj + mjk * uk)
                vkb[sl] = -(mki * ui + mjk * uj + mkk * uk)
            pltpu.sync_copy(vib, acc_sp.at[iv], add=True)
            pltpu.sync_copy(vjb, acc_sp.at[jv], add=True)
            pltpu.sync_copy(vkb, acc_sp.at[kv], add=True)
            return carry

        lax.fori_loop(0, nblk, blk, 0)
        plsc.subcore_barrier()
        pltpu.sync_copy(acc_sp.at[pl.ds(noff, chunk)],
                        out_h.at[pl.ds(c * npad + noff, chunk)])

    return body


def _tc_sum(a_ref, o_ref):
    o_ref[...] = a_ref[0] + a_ref[1]


def kernel(x_comp, u, cell_node_map):
    n = x_comp.shape[0]
    t = cell_node_map.shape[0]
    nblk = -(-t // (NW * B))
    tpad = NW * B * nblk
    npad = -(-(n + 1) // 2048) * 2048
    chunk = npad // NS

    xs = jnp.pad(x_comp[:, 0], (0, npad - n))
    ys = jnp.pad(x_comp[:, 1], (0, npad - n))
    us = jnp.pad(u, (0, npad - n))
    zs = jnp.zeros((npad,), jnp.float32)
    pad_cfg = ((0, tpad - t),)
    i1 = jnp.pad(cell_node_map[:, 0], pad_cfg, constant_values=n)
    j1 = jnp.pad(cell_node_map[:, 1], pad_cfg, constant_values=n)
    k1 = jnp.pad(cell_node_map[:, 2], pad_cfg, constant_values=n)

    partials = _sc_assemble(npad, nblk, chunk)(xs, ys, us, zs, i1, j1, k1)

    summed = pl.pallas_call(
        _tc_sum,
        out_shape=jax.ShapeDtypeStruct((npad // 128, 128), jnp.float32),
    )(partials.reshape(NC, npad // 128, 128))
    return summed.reshape(npad)[:n]
